# baseline (speedup 1.0000x reference)
import jax
import jax.numpy as jnp
from jax.experimental import pallas as pl
from jax.experimental.pallas import tpu as pltpu

ALPHA = 8
NCHUNK = 8


def _pack_body(x_hbm, slow_hbm, fast_hbm, *rest):
    bufs = rest[:NCHUNK]
    in_sems = rest[NCHUNK:2 * NCHUNK]
    out_sem = rest[2 * NCHUNK]
    slow_sem = rest[2 * NCHUNK + 1]

    in_cps = []
    for k in range(NCHUNK):
        cp = pltpu.make_async_copy(
            x_hbm.at[:, pl.ds(ALPHA * k, ALPHA)], bufs[k], in_sems[k])
        cp.start()
        in_cps.append(cp)
    out_cps = []
    for k in range(NCHUNK):
        in_cps[k].wait()
        fast_cp = pltpu.make_async_copy(
            bufs[k], fast_hbm.at[:, pl.ds(ALPHA * k, ALPHA)], out_sem)
        fast_cp.start()
        slow_cp = pltpu.make_async_copy(
            bufs[k].at[:, k], slow_hbm.at[:, k], slow_sem)
        slow_cp.start()
        out_cps.append((fast_cp, slow_cp))
    for fast_cp, slow_cp in out_cps:
        fast_cp.wait()
        slow_cp.wait()


def kernel(x):
    C, T, H, W = x.shape
    G = T // ALPHA
    slow, fast = pl.pallas_call(
        _pack_body,
        in_specs=[pl.BlockSpec(memory_space=pl.ANY)],
        out_specs=[
            pl.BlockSpec(memory_space=pl.ANY),
            pl.BlockSpec(memory_space=pl.ANY),
        ],
        out_shape=[
            jax.ShapeDtypeStruct((C, G, H, W), x.dtype),
            jax.ShapeDtypeStruct((C, T, H, W), x.dtype),
        ],
        scratch_shapes=(
            [pltpu.VMEM((C, ALPHA, H, W), x.dtype) for _ in range(NCHUNK)]
            + [pltpu.SemaphoreType.DMA for _ in range(NCHUNK)]
            + [pltpu.SemaphoreType.DMA, pltpu.SemaphoreType.DMA]
        ),
    )(x)
    return (slow, fast)


# speedup vs baseline: 1.0126x; 1.0126x over previous
import jax
import jax.numpy as jnp
from jax.experimental import pallas as pl
from jax.experimental.pallas import tpu as pltpu

ALPHA = 8
NCHUNK = 4
CHUNK = 16


def _pack_body(x_hbm, slow_hbm, fast_hbm, *rest):
    bufs = rest[:NCHUNK]
    in_sems = rest[NCHUNK:2 * NCHUNK]
    out_sem = rest[2 * NCHUNK]
    slow_sem = rest[2 * NCHUNK + 1]

    in_cps = []
    for k in range(NCHUNK):
        cp = pltpu.make_async_copy(
            x_hbm.at[:, pl.ds(CHUNK * k, CHUNK)], bufs[k], in_sems[k])
        cp.start()
        in_cps.append(cp)
    out_cps = []
    for k in range(NCHUNK):
        in_cps[k].wait()
        fast_cp = pltpu.make_async_copy(
            bufs[k], fast_hbm.at[:, pl.ds(CHUNK * k, CHUNK)], out_sem)
        fast_cp.start()
        out_cps.append(fast_cp)
        for j in range(2):
            s = 2 * k + j
            slow_cp = pltpu.make_async_copy(
                bufs[k].at[:, 2 * k + 9 * j], slow_hbm.at[:, s], slow_sem)
            slow_cp.start()
            out_cps.append(slow_cp)
    for cp in out_cps:
        cp.wait()


def kernel(x):
    C, T, H, W = x.shape
    G = T // ALPHA
    slow, fast = pl.pallas_call(
        _pack_body,
        in_specs=[pl.BlockSpec(memory_space=pl.ANY)],
        out_specs=[
            pl.BlockSpec(memory_space=pl.ANY),
            pl.BlockSpec(memory_space=pl.ANY),
        ],
        out_shape=[
            jax.ShapeDtypeStruct((C, G, H, W), x.dtype),
            jax.ShapeDtypeStruct((C, T, H, W), x.dtype),
        ],
        scratch_shapes=(
            [pltpu.VMEM((C, CHUNK, H, W), x.dtype) for _ in range(NCHUNK)]
            + [pltpu.SemaphoreType.DMA for _ in range(NCHUNK)]
            + [pltpu.SemaphoreType.DMA, pltpu.SemaphoreType.DMA]
        ),
    )(x)
    return (slow, fast)


# speedup vs baseline: 1.0136x; 1.0010x over previous
import jax
from jax.experimental import pallas as pl
from jax.experimental.pallas import tpu as pltpu

ALPHA = 8
NCHUNK = 4
CHUNK = 16


def _pack_body(x_hbm, slow_hbm, fast_hbm, *rest):
    bufs = rest[:NCHUNK]
    in_sems = rest[NCHUNK:2 * NCHUNK]
    out_sem = rest[2 * NCHUNK]
    slow_sem = rest[2 * NCHUNK + 1]

    in_cps = []
    for k in range(NCHUNK):
        cp = pltpu.make_async_copy(
            x_hbm.at[:, pl.ds(CHUNK * k, CHUNK)], bufs[k], in_sems[k])
        cp.start()
        in_cps.append(cp)
    out_cps = []
    for k in range(NCHUNK):
        in_cps[k].wait()
        fast_cp = pltpu.make_async_copy(
            bufs[k], fast_hbm.at[:, pl.ds(CHUNK * k, CHUNK)], out_sem)
        fast_cp.start()
        out_cps.append(fast_cp)
        for j in range(2):
            s = 2 * k + j
            slow_cp = pltpu.make_async_copy(
                bufs[k].at[:, 2 * k + 9 * j], slow_hbm.at[:, s], slow_sem)
            slow_cp.start()
            out_cps.append(slow_cp)
    for cp in out_cps:
        cp.wait()


def kernel(x):
    C, T, H, W = x.shape
    G = T // ALPHA
    slow, fast = pl.pallas_call(
        _pack_body,
        in_specs=[pl.BlockSpec(memory_space=pl.ANY)],
        out_specs=[
            pl.BlockSpec(memory_space=pl.ANY),
            pl.BlockSpec(memory_space=pl.ANY),
        ],
        out_shape=[
            jax.ShapeDtypeStruct((C, G, H, W), x.dtype),
            jax.ShapeDtypeStruct((C, T, H, W), x.dtype),
        ],
        scratch_shapes=(
            [pltpu.VMEM((C, CHUNK, H, W), x.dtype) for _ in range(NCHUNK)]
            + [pltpu.SemaphoreType.DMA for _ in range(NCHUNK)]
            + [pltpu.SemaphoreType.DMA, pltpu.SemaphoreType.DMA]
        ),
    )(x)
    return (slow, fast)


# speedup vs baseline: 1.0233x; 1.0096x over previous
import jax
from jax.experimental import pallas as pl
from jax.experimental.pallas import tpu as pltpu

ALPHA = 8
NCHUNK = 2
CHUNK = 32


def _pack_body(x_hbm, slow_hbm, fast_hbm, *rest):
    bufs = rest[:NCHUNK]
    in_sems = rest[NCHUNK:2 * NCHUNK]
    out_sem = rest[2 * NCHUNK]
    slow_sem = rest[2 * NCHUNK + 1]

    in_cps = []
    for k in range(NCHUNK):
        cp = pltpu.make_async_copy(
            x_hbm.at[:, pl.ds(CHUNK * k, CHUNK)], bufs[k], in_sems[k])
        cp.start()
        in_cps.append(cp)
    out_cps = []
    for k in range(NCHUNK):
        in_cps[k].wait()
        fast_cp = pltpu.make_async_copy(
            bufs[k], fast_hbm.at[:, pl.ds(CHUNK * k, CHUNK)], out_sem)
        fast_cp.start()
        out_cps.append(fast_cp)
        for j in range(4):
            s = 4 * k + j
            slow_cp = pltpu.make_async_copy(
                bufs[k].at[:, 4 * k + 9 * j], slow_hbm.at[:, s], slow_sem)
            slow_cp.start()
            out_cps.append(slow_cp)
    for cp in out_cps:
        cp.wait()


def kernel(x):
    C, T, H, W = x.shape
    G = T // ALPHA
    slow, fast = pl.pallas_call(
        _pack_body,
        in_specs=[pl.BlockSpec(memory_space=pl.ANY)],
        out_specs=[
            pl.BlockSpec(memory_space=pl.ANY),
            pl.BlockSpec(memory_space=pl.ANY),
        ],
        out_shape=[
            jax.ShapeDtypeStruct((C, G, H, W), x.dtype),
            jax.ShapeDtypeStruct((C, T, H, W), x.dtype),
        ],
        scratch_shapes=(
            [pltpu.VMEM((C, CHUNK, H, W), x.dtype) for _ in range(NCHUNK)]
            + [pltpu.SemaphoreType.DMA for _ in range(NCHUNK)]
            + [pltpu.SemaphoreType.DMA, pltpu.SemaphoreType.DMA]
        ),
    )(x)
    return (slow, fast)
